# baseline (speedup 1.0000x reference)
import functools

import jax
import jax.numpy as jnp
from jax import lax
from jax.experimental import pallas as pl
from jax.experimental.pallas import tpu as pltpu
from jax.experimental.pallas import tpu_sc as plsc

NC = 2
NS = 16
L = 16
CHUNK = 128


def _bias_mf_body(latdim, b_per_w, u_hbm, i_hbm, ub_hbm, ib_hbm, usr_hbm,
                  itm_hbm, out_hbm, usr_v, itm_v, urows, irows, ubv, ibv,
                  outv, sem):
  wid = lax.axis_index("s") * NC + lax.axis_index("c")
  base = wid * b_per_w

  pltpu.sync_copy(usr_hbm.at[pl.ds(base, b_per_w)], usr_v)
  pltpu.sync_copy(itm_hbm.at[pl.ds(base, b_per_w)], itm_v)

  copies = []
  for g in range(b_per_w // CHUNK):
    sl = pl.ds(g * CHUNK, CHUNK)
    copies.append(pltpu.async_copy(u_hbm.at[usr_v.at[sl]], urows.at[sl], sem))
    copies.append(pltpu.async_copy(i_hbm.at[itm_v.at[sl]], irows.at[sl], sem))
    copies.append(pltpu.async_copy(ub_hbm.at[usr_v.at[sl]], ubv.at[sl], sem))
    copies.append(pltpu.async_copy(ib_hbm.at[itm_v.at[sl]], ibv.at[sl], sem))
  for c in copies:
    c.wait()

  nvec = latdim // L
  lane = lax.iota(jnp.int32, L)
  dnums = lax.GatherDimensionNumbers(
      offset_dims=(), collapsed_slice_dims=(0,), start_index_map=(0,))

  def shufxor(x, k):
    return lax.gather(x, (lane ^ k)[:, None], dnums, (1,),
                      mode=lax.GatherScatterMode.PROMISE_IN_BOUNDS)

  def group(g, carry):
    base_r = g * L
    vecs = []
    for j in range(L):
      r = base_r + j
      acc = urows[r, pl.ds(0, L)] * irows[r, pl.ds(0, L)]
      for c in range(1, nvec):
        acc = acc + urows[r, pl.ds(c * L, L)] * irows[r, pl.ds(c * L, L)]
      vecs.append(acc)
    for k in (1, 2, 4, 8):
      nxt = []
      sel = (lane & k) == 0
      for p in range(0, len(vecs), 2):
        a, b = vecs[p], vecs[p + 1]
        nxt.append(jnp.where(sel, a + shufxor(a, k), b + shufxor(b, k)))
      vecs = nxt
    sl = pl.ds(base_r, L)
    outv[sl] = vecs[0] + ubv[sl] + ibv[sl]
    return carry

  lax.fori_loop(0, b_per_w // L, group, None)

  pltpu.sync_copy(outv, out_hbm.at[pl.ds(base, b_per_w)])


def kernel(uEmbeds, iEmbeds, uBias, iBias, usr, itm):
  batch = usr.shape[0]
  latdim = uEmbeds.shape[1]
  nw = NC * NS
  b_per_w = batch // nw
  mesh = plsc.VectorSubcoreMesh(
      core_axis_name="c", subcore_axis_name="s", num_cores=NC,
      num_subcores=NS)
  k = pl.kernel(
      functools.partial(_bias_mf_body, latdim, b_per_w),
      out_type=jax.ShapeDtypeStruct((batch,), jnp.float32),
      mesh=mesh,
      scratch_types=[
          pltpu.VMEM((b_per_w,), jnp.int32),
          pltpu.VMEM((b_per_w,), jnp.int32),
          pltpu.VMEM((b_per_w, latdim), jnp.float32),
          pltpu.VMEM((b_per_w, latdim), jnp.float32),
          pltpu.VMEM((b_per_w,), jnp.float32),
          pltpu.VMEM((b_per_w,), jnp.float32),
          pltpu.VMEM((b_per_w,), jnp.float32),
          pltpu.SemaphoreType.DMA,
      ],
      compiler_params=pltpu.CompilerParams(use_tc_tiling_on_sc=False),
  )
  return k(uEmbeds, iEmbeds, uBias, iBias, usr, itm)


# speedup vs baseline: 2.1647x; 2.1647x over previous
import functools

import jax
import jax.numpy as jnp
from jax import lax
from jax.experimental import pallas as pl
from jax.experimental.pallas import tpu as pltpu
from jax.experimental.pallas import tpu_sc as plsc

NC = 2
NS = 16
L = 16
CHUNK = 128
W = 128


def _bias_mf_body(latdim, b_per_w, ut_hbm, it_hbm, ub_hbm, ib_hbm, usr_hbm,
                  itm_hbm, out_hbm, usr_v, itm_v, ublk, iblk, ubv, ibv,
                  outv, sem_a, sem_b, bsem):
  wid = lax.axis_index("s") * NC + lax.axis_index("c")
  base = wid * b_per_w
  n_groups = b_per_w // L

  pltpu.sync_copy(usr_hbm.at[pl.ds(base, b_per_w)], usr_v)
  pltpu.sync_copy(itm_hbm.at[pl.ds(base, b_per_w)], itm_v)

  bias_copies = []
  for g in range(b_per_w // CHUNK):
    sl = pl.ds(g * CHUNK, CHUNK)
    bias_copies.append(
        pltpu.async_copy(ub_hbm.at[usr_v.at[sl]], ubv.at[sl], bsem))
    bias_copies.append(
        pltpu.async_copy(ib_hbm.at[itm_v.at[sl]], ibv.at[sl], bsem))

  lane = lax.iota(jnp.int32, L)
  nvec = latdim // L
  dnums = lax.GatherDimensionNumbers(
      offset_dims=(), collapsed_slice_dims=(0,), start_index_map=(0,))

  def shufxor(x, k):
    return lax.gather(x, (lane ^ k)[:, None], dnums, (1,),
                      mode=lax.GatherScatterMode.PROMISE_IN_BOUNDS)

  def fire(u_idx, i_idx, buf, sem):
    ua = pl.multiple_of(u_idx & ~(W - 1), W)
    ia = pl.multiple_of(i_idx & ~(W - 1), W)
    pltpu.async_copy(ut_hbm.at[:, pl.ds(ua, W)], ublk.at[buf], sem)
    pltpu.async_copy(it_hbm.at[:, pl.ds(ia, W)], iblk.at[buf], sem)

  def drain(buf, sem):
    pltpu.make_async_copy(
        ut_hbm.at[:, pl.ds(0, W)], ublk.at[buf], sem).wait()
    pltpu.make_async_copy(
        it_hbm.at[:, pl.ds(0, W)], iblk.at[buf], sem).wait()

  def pair_dot(buf, uoff, ioff):
    bufv = jnp.full((L,), buf, jnp.int32)
    uof = jnp.full((L,), uoff, jnp.int32)
    iof = jnp.full((L,), ioff, jnp.int32)
    acc = jnp.zeros((L,), jnp.float32)
    for c in range(nvec):
      fv = c * L + lane
      uv = plsc.load_gather(ublk, [bufv, fv, uof])
      iv = plsc.load_gather(iblk, [bufv, fv, iof])
      acc = acc + uv * iv
    for k in (1, 2, 4, 8):
      acc = acc + shufxor(acc, k)
    return acc

  for c in bias_copies:
    c.wait()

  uvec0 = usr_v[pl.ds(0, L)]
  ivec0 = itm_v[pl.ds(0, L)]
  fire(uvec0[0], ivec0[0], 0, sem_a)

  def group(g, carry):
    uvec, ivec = carry
    sl = pl.ds(g * L, L)
    vec = ubv[sl] + ibv[sl]
    nuvec, nivec = uvec, ivec
    for j in range(L):
      buf = j % 2
      sem = sem_a if buf == 0 else sem_b
      nsem = sem_b if buf == 0 else sem_a
      if j < L - 1:
        fire(uvec[j + 1], ivec[j + 1], 1 - buf, nsem)
      else:
        gn = jnp.minimum(g + 1, n_groups - 1)
        nsl = pl.ds(gn * L, L)
        nuvec = usr_v[nsl]
        nivec = itm_v[nsl]
        fire(nuvec[0], nivec[0], 1 - buf, nsem)
      drain(buf, sem)
      tot = pair_dot(buf, uvec[j] & (W - 1), ivec[j] & (W - 1))
      vec = jnp.where(lane == j, vec + tot, vec)
    outv[sl] = vec
    return (nuvec, nivec)

  lax.fori_loop(0, n_groups, group, (uvec0, ivec0))
  drain(0, sem_a)

  pltpu.sync_copy(outv, out_hbm.at[pl.ds(base, b_per_w)])


def kernel(uEmbeds, iEmbeds, uBias, iBias, usr, itm):
  batch = usr.shape[0]
  latdim = uEmbeds.shape[1]
  nw = NC * NS
  b_per_w = batch // nw
  uT = uEmbeds.T
  iT = iEmbeds.T
  mesh = plsc.VectorSubcoreMesh(
      core_axis_name="c", subcore_axis_name="s", num_cores=NC,
      num_subcores=NS)
  k = pl.kernel(
      functools.partial(_bias_mf_body, latdim, b_per_w),
      out_type=jax.ShapeDtypeStruct((batch,), jnp.float32),
      mesh=mesh,
      scratch_types=[
          pltpu.VMEM((b_per_w,), jnp.int32),
          pltpu.VMEM((b_per_w,), jnp.int32),
          pltpu.VMEM((2, latdim, W), jnp.float32),
          pltpu.VMEM((2, latdim, W), jnp.float32),
          pltpu.VMEM((b_per_w,), jnp.float32),
          pltpu.VMEM((b_per_w,), jnp.float32),
          pltpu.VMEM((b_per_w,), jnp.float32),
          pltpu.SemaphoreType.DMA,
          pltpu.SemaphoreType.DMA,
          pltpu.SemaphoreType.DMA,
      ],
      compiler_params=pltpu.CompilerParams(
          use_tc_tiling_on_sc=True, needs_layout_passes=False),
  )
  return k(uT, iT, uBias, iBias, usr, itm)


# speedup vs baseline: 2.6160x; 1.2085x over previous
import functools

import jax
import jax.numpy as jnp
from jax import lax
from jax.experimental import pallas as pl
from jax.experimental.pallas import tpu as pltpu
from jax.experimental.pallas import tpu_sc as plsc

NC = 2
NS = 16
L = 16
CHUNK = 128
W = 128


NBUF = 4


def _bias_mf_body(latdim, b_per_w, ut_hbm, it_hbm, ub_hbm, ib_hbm, usr_hbm,
                  itm_hbm, out_hbm, usr_v, itm_v, ublk, iblk, ubv, ibv,
                  outv, sem0, sem1, sem2, sem3, bsem):
  sems = (sem0, sem1, sem2, sem3)
  wid = lax.axis_index("s") * NC + lax.axis_index("c")
  base = wid * b_per_w
  n_groups = b_per_w // L

  pltpu.sync_copy(usr_hbm.at[pl.ds(base, b_per_w)], usr_v)
  pltpu.sync_copy(itm_hbm.at[pl.ds(base, b_per_w)], itm_v)

  bias_copies = []
  for g in range(b_per_w // CHUNK):
    sl = pl.ds(g * CHUNK, CHUNK)
    bias_copies.append(
        pltpu.async_copy(ub_hbm.at[usr_v.at[sl]], ubv.at[sl], bsem))
    bias_copies.append(
        pltpu.async_copy(ib_hbm.at[itm_v.at[sl]], ibv.at[sl], bsem))

  lane = lax.iota(jnp.int32, L)
  nvec = latdim // L
  dnums = lax.GatherDimensionNumbers(
      offset_dims=(), collapsed_slice_dims=(0,), start_index_map=(0,))

  def shufxor(x, k):
    return lax.gather(x, (lane ^ k)[:, None], dnums, (1,),
                      mode=lax.GatherScatterMode.PROMISE_IN_BOUNDS)

  def fire(u_idx, i_idx, buf, sem):
    ua = pl.multiple_of(u_idx & ~(W - 1), W)
    ia = pl.multiple_of(i_idx & ~(W - 1), W)
    pltpu.async_copy(ut_hbm.at[:, pl.ds(ua, W)], ublk.at[buf], sem)
    pltpu.async_copy(it_hbm.at[:, pl.ds(ia, W)], iblk.at[buf], sem)

  def drain(buf, sem):
    pltpu.make_async_copy(
        ut_hbm.at[:, pl.ds(0, W)], ublk.at[buf], sem).wait()
    pltpu.make_async_copy(
        it_hbm.at[:, pl.ds(0, W)], iblk.at[buf], sem).wait()

  def pair_dot(buf, uoff, ioff):
    bufv = jnp.full((L,), buf, jnp.int32)
    uof = jnp.full((L,), uoff, jnp.int32)
    iof = jnp.full((L,), ioff, jnp.int32)
    acc = jnp.zeros((L,), jnp.float32)
    for c in range(nvec):
      fv = c * L + lane
      uv = plsc.load_gather(ublk, [bufv, fv, uof])
      iv = plsc.load_gather(iblk, [bufv, fv, iof])
      acc = acc + uv * iv
    for k in (1, 2, 4, 8):
      acc = acc + shufxor(acc, k)
    return acc

  for c in bias_copies:
    c.wait()

  depth = NBUF - 1
  uvec0 = usr_v[pl.ds(0, L)]
  ivec0 = itm_v[pl.ds(0, L)]
  for b in range(depth):
    fire(uvec0[b], ivec0[b], b, sems[b])

  def group(g, carry):
    uvec, ivec = carry
    sl = pl.ds(g * L, L)
    vec = ubv[sl] + ibv[sl]
    gn = jnp.minimum(g + 1, n_groups - 1)
    nsl = pl.ds(gn * L, L)
    nuvec = usr_v[nsl]
    nivec = itm_v[nsl]
    for j in range(L):
      buf = j % NBUF
      jn = j + depth
      fvec, gvec = (uvec, ivec) if jn < L else (nuvec, nivec)
      fire(fvec[jn % L], gvec[jn % L], jn % NBUF, sems[jn % NBUF])
      drain(buf, sems[buf])
      tot = pair_dot(buf, uvec[j] & (W - 1), ivec[j] & (W - 1))
      vec = jnp.where(lane == j, vec + tot, vec)
    outv[sl] = vec
    return (nuvec, nivec)

  lax.fori_loop(0, n_groups, group, (uvec0, ivec0))
  for b in range(depth):
    drain(b % NBUF, sems[b % NBUF])

  pltpu.sync_copy(outv, out_hbm.at[pl.ds(base, b_per_w)])


def kernel(uEmbeds, iEmbeds, uBias, iBias, usr, itm):
  batch = usr.shape[0]
  latdim = uEmbeds.shape[1]
  nw = NC * NS
  b_per_w = batch // nw
  uT = uEmbeds.T
  iT = iEmbeds.T
  mesh = plsc.VectorSubcoreMesh(
      core_axis_name="c", subcore_axis_name="s", num_cores=NC,
      num_subcores=NS)
  k = pl.kernel(
      functools.partial(_bias_mf_body, latdim, b_per_w),
      out_type=jax.ShapeDtypeStruct((batch,), jnp.float32),
      mesh=mesh,
      scratch_types=[
          pltpu.VMEM((b_per_w,), jnp.int32),
          pltpu.VMEM((b_per_w,), jnp.int32),
          pltpu.VMEM((NBUF, latdim, W), jnp.float32),
          pltpu.VMEM((NBUF, latdim, W), jnp.float32),
          pltpu.VMEM((b_per_w,), jnp.float32),
          pltpu.VMEM((b_per_w,), jnp.float32),
          pltpu.VMEM((b_per_w,), jnp.float32),
          pltpu.SemaphoreType.DMA,
          pltpu.SemaphoreType.DMA,
          pltpu.SemaphoreType.DMA,
          pltpu.SemaphoreType.DMA,
          pltpu.SemaphoreType.DMA,
      ],
      compiler_params=pltpu.CompilerParams(
          use_tc_tiling_on_sc=True, needs_layout_passes=False),
  )
  return k(uT, iT, uBias, iBias, usr, itm)
